# Initial kernel scaffold; baseline (speedup 1.0000x reference)
import jax
import jax.numpy as jnp
from jax.experimental import pallas as pl


def kernel(x, edge_index, edge_attr, W1, b1, W2, b2):
    raise NotImplementedError("write your pallas kernel here")


# speedup vs baseline: 14.3060x; 14.3060x over previous
import functools

import jax
import jax.numpy as jnp
from jax import lax
from jax.experimental import pallas as pl
from jax.experimental.pallas import tpu as pltpu
from jax.experimental.pallas import tpu_sc as plsc

NC = 2
NS = 16
NW = NC * NS
LANES = 16
CHUNK = 128


def _deg_body(dst_hbm, w_hbm, out_hbm, dst_v, w_v, zero_v, deg_sh, sem,
              *, n_pad, nchunks):
    cid = lax.axis_index("c")
    sid = lax.axis_index("s")
    wid = sid * NC + cid
    rps = n_pad // NS
    row0 = sid * rps

    copy_d = pltpu.async_copy(dst_hbm.at[wid], dst_v, sem)
    copy_w = pltpu.async_copy(w_hbm.at[wid], w_v, sem)

    def zero_body(i, _):
        zero_v[pl.ds(i * LANES, LANES)] = jnp.zeros((LANES,), jnp.float32)
        return 0

    lax.fori_loop(0, rps // LANES, zero_body, 0)
    pltpu.sync_copy(zero_v, deg_sh.at[pl.ds(row0, rps)])
    copy_d.wait()
    copy_w.wait()
    plsc.subcore_barrier()

    def acc_body(c, _):
        pltpu.sync_copy(w_v.at[c], deg_sh.at[dst_v.at[c]], add=True)
        return 0

    lax.fori_loop(0, nchunks, acc_body, 0)
    plsc.subcore_barrier()
    pltpu.sync_copy(deg_sh.at[pl.ds(row0, rps)], zero_v)
    pltpu.sync_copy(zero_v, out_hbm.at[cid, 0, pl.ds(row0, rps)])


def _deg_partials(dst, w, n_pad, nchunks):
    mesh = plsc.VectorSubcoreMesh(core_axis_name="c", subcore_axis_name="s")
    kern = pl.kernel(
        functools.partial(_deg_body, n_pad=n_pad, nchunks=nchunks),
        out_type=jax.ShapeDtypeStruct((NC, 1, n_pad), jnp.float32),
        mesh=mesh,
        scratch_types=[
            pltpu.VMEM((nchunks, CHUNK), jnp.int32),
            pltpu.VMEM((nchunks, CHUNK), jnp.float32),
            pltpu.VMEM((n_pad // NS,), jnp.float32),
            pltpu.VMEM_SHARED((n_pad,), jnp.float32),
            pltpu.SemaphoreType.DMA,
        ],
    )
    return kern(dst, w)


def _spmm_body(src_hbm, dst_hbm, w_hbm, g_hbm, out_hbm,
               src_v, dst_v, w_v, rows_v, acc_sh, sem, *, n_pad, d, nchunks):
    cid = lax.axis_index("c")
    sid = lax.axis_index("s")
    wid = sid * NC + cid
    rps = n_pad // NS
    row0 = sid * rps

    copy_s = pltpu.async_copy(src_hbm.at[wid], src_v, sem)
    copy_d = pltpu.async_copy(dst_hbm.at[wid], dst_v, sem)
    copy_w = pltpu.async_copy(w_hbm.at[wid], w_v, sem)

    def zrow(i, _):
        rows_v[pl.ds(i, 1), :] = jnp.zeros((1, d), jnp.float32)
        return 0

    lax.fori_loop(0, CHUNK, zrow, 0)
    for k in range(rps // CHUNK):
        pltpu.sync_copy(rows_v, acc_sh.at[pl.ds(row0 + k * CHUNK, CHUNK)])
    copy_s.wait()
    copy_d.wait()
    copy_w.wait()
    plsc.subcore_barrier()

    def chunk_body(c, _):
        pltpu.async_copy(g_hbm.at[src_v.at[c]], rows_v, sem).wait()

        def scale(g, _):
            w16 = w_v[c, pl.ds(g * LANES, LANES)]
            base = g * LANES
            for jj in range(LANES):
                wj = w16[jj]
                for k in range(d // LANES):
                    sl = pl.ds(k * LANES, LANES)
                    rows_v[base + jj, sl] = rows_v[base + jj, sl] * wj
            return 0

        lax.fori_loop(0, CHUNK // LANES, scale, 0)
        pltpu.sync_copy(rows_v, acc_sh.at[dst_v.at[c]], add=True)
        return 0

    lax.fori_loop(0, nchunks, chunk_body, 0)
    plsc.subcore_barrier()

    for k in range(rps // CHUNK):
        r = row0 + k * CHUNK
        pltpu.sync_copy(acc_sh.at[pl.ds(r, CHUNK)], rows_v)
        pltpu.sync_copy(rows_v, out_hbm.at[cid, pl.ds(r, CHUNK)])


def _spmm_partials(src, dst, w, g, n_pad, d, nchunks):
    mesh = plsc.VectorSubcoreMesh(core_axis_name="c", subcore_axis_name="s")
    kern = pl.kernel(
        functools.partial(_spmm_body, n_pad=n_pad, d=d, nchunks=nchunks),
        out_type=jax.ShapeDtypeStruct((NC, n_pad, d), jnp.float32),
        mesh=mesh,
        compiler_params=pltpu.CompilerParams(use_tc_tiling_on_sc=False),
        scratch_types=[
            pltpu.VMEM((nchunks, CHUNK), jnp.int32),
            pltpu.VMEM((nchunks, CHUNK), jnp.int32),
            pltpu.VMEM((nchunks, CHUNK), jnp.float32),
            pltpu.VMEM((CHUNK, d), jnp.float32),
            pltpu.VMEM_SHARED((n_pad, d), jnp.float32),
            pltpu.SemaphoreType.DMA,
        ],
    )
    return kern(src, dst, w, g)


def _prep_body(degp_ref, x_ref, w1_ref, g1_ref, dinv_ref):
    deg = 1.0 + jnp.sum(degp_ref[...], axis=1)
    dinv = jnp.where(deg > 0, lax.rsqrt(deg), 0.0)
    dinv_ref[...] = dinv[:, None]
    h = jnp.dot(x_ref[...], w1_ref[...], preferred_element_type=jnp.float32)
    g1_ref[...] = h * dinv[:, None]


def _prep(degp, x, w1, n, blk):
    d_in, h_dim = x.shape[1], w1.shape[1]
    grid = (n // blk,)
    return pl.pallas_call(
        _prep_body,
        grid=grid,
        in_specs=[
            pl.BlockSpec((blk, NC), lambda i: (i, 0)),
            pl.BlockSpec((blk, d_in), lambda i: (i, 0)),
            pl.BlockSpec((d_in, h_dim), lambda i: (0, 0)),
        ],
        out_specs=[
            pl.BlockSpec((blk, h_dim), lambda i: (i, 0)),
            pl.BlockSpec((blk, 1), lambda i: (i, 0)),
        ],
        out_shape=[
            jax.ShapeDtypeStruct((n, h_dim), jnp.float32),
            jax.ShapeDtypeStruct((n, 1), jnp.float32),
        ],
    )(degp, x, w1)


def _mid_body(aggp_ref, g1_ref, dinv_ref, b1_ref, w2_ref, g2_ref):
    agg = aggp_ref[0] + aggp_ref[1] + g1_ref[...]
    t = jnp.maximum(agg * dinv_ref[...] + b1_ref[...], 0.0)
    h2 = jnp.dot(t, w2_ref[...], preferred_element_type=jnp.float32)
    g2_ref[...] = h2 * dinv_ref[...]


def _mid(aggp, g1, dinv, b1, w2p, n, blk):
    h_dim, c_pad = w2p.shape
    return pl.pallas_call(
        _mid_body,
        grid=(n // blk,),
        in_specs=[
            pl.BlockSpec((NC, blk, h_dim), lambda i: (0, i, 0)),
            pl.BlockSpec((blk, h_dim), lambda i: (i, 0)),
            pl.BlockSpec((blk, 1), lambda i: (i, 0)),
            pl.BlockSpec((1, h_dim), lambda i: (0, 0)),
            pl.BlockSpec((h_dim, c_pad), lambda i: (0, 0)),
        ],
        out_specs=pl.BlockSpec((blk, c_pad), lambda i: (i, 0)),
        out_shape=jax.ShapeDtypeStruct((n, c_pad), jnp.float32),
    )(aggp, g1, dinv, b1[None, :], w2p)


def _fin_body(aggp_ref, g2_ref, dinv_ref, b2_ref, out_ref, *, c):
    agg = aggp_ref[0] + aggp_ref[1] + g2_ref[...]
    z = (agg * dinv_ref[...] + b2_ref[...])[:, :c]
    m = jnp.max(z, axis=1, keepdims=True)
    lse = jnp.log(jnp.sum(jnp.exp(z - m), axis=1, keepdims=True))
    out_ref[...] = z - m - lse


def _fin(aggp, g2, dinv, b2p, n, c, blk):
    c_pad = g2.shape[1]
    return pl.pallas_call(
        functools.partial(_fin_body, c=c),
        grid=(n // blk,),
        in_specs=[
            pl.BlockSpec((NC, blk, c_pad), lambda i: (0, i, 0)),
            pl.BlockSpec((blk, c_pad), lambda i: (i, 0)),
            pl.BlockSpec((blk, 1), lambda i: (i, 0)),
            pl.BlockSpec((1, c_pad), lambda i: (0, 0)),
        ],
        out_specs=pl.BlockSpec((blk, c), lambda i: (i, 0)),
        out_shape=jax.ShapeDtypeStruct((n, c), jnp.float32),
    )(aggp, g2, dinv, b2p[None, :])


def kernel(x, edge_index, edge_attr, W1, b1, W2, b2):
    n, d_in = x.shape
    e = edge_attr.shape[0]
    h_dim = W1.shape[1]
    c = W2.shape[1]
    c_pad = ((c + LANES - 1) // LANES) * LANES
    n_pad = ((n + NS * CHUNK - 1) // (NS * CHUNK)) * (NS * CHUNK)

    epw = ((e + NW * CHUNK - 1) // (NW * CHUNK)) * CHUNK
    e_pad = epw * NW
    nchunks = epw // CHUNK
    pad = e_pad - e
    src = jnp.concatenate([edge_index[0], jnp.zeros((pad,), jnp.int32)])
    dst = jnp.concatenate([edge_index[1], jnp.zeros((pad,), jnp.int32)])
    w = jnp.concatenate([edge_attr, jnp.zeros((pad,), jnp.float32)])
    src3 = src.reshape(NW, nchunks, CHUNK)
    dst3 = dst.reshape(NW, nchunks, CHUNK)
    w3 = w.reshape(NW, nchunks, CHUNK)

    degp = _deg_partials(dst3, w3, n_pad, nchunks)[:, 0, :n].T

    blk = 1000 if n % 1000 == 0 else n // 8
    g1, dinv = _prep(degp, x, W1, n, blk)

    agg1 = _spmm_partials(src3, dst3, w3, g1, n_pad, h_dim, nchunks)[:, :n]

    w2p = jnp.pad(W2, ((0, 0), (0, c_pad - c)))
    b2p = jnp.pad(b2, (0, c_pad - c))
    g2 = _mid(agg1, g1, dinv, b1, w2p, n, blk)

    agg2 = _spmm_partials(src3, dst3, w3, g2, n_pad, c_pad, nchunks)[:, :n]

    return _fin(agg2, g2, dinv, b2p, n, c, blk)


# speedup vs baseline: 16.6690x; 1.1652x over previous
import functools

import jax
import jax.numpy as jnp
from jax import lax
from jax.experimental import pallas as pl
from jax.experimental.pallas import tpu as pltpu
from jax.experimental.pallas import tpu_sc as plsc

NC = 2
NS = 16
NW = NC * NS
LANES = 16
CHUNK = 128
NBUF = 4


def _deg_body(dst_hbm, w_hbm, out_hbm, dst_v, w_v, zero_v, deg_sh, sem,
              *, n_pad, nchunks):
    cid = lax.axis_index("c")
    sid = lax.axis_index("s")
    wid = sid * NC + cid
    rps = n_pad // NS
    row0 = sid * rps

    copy_d = pltpu.async_copy(dst_hbm.at[wid], dst_v, sem)
    copy_w = pltpu.async_copy(w_hbm.at[wid], w_v, sem)

    def zero_body(i, _):
        zero_v[pl.ds(i * LANES, LANES)] = jnp.zeros((LANES,), jnp.float32)
        return 0

    lax.fori_loop(0, rps // LANES, zero_body, 0)
    pltpu.sync_copy(zero_v, deg_sh.at[pl.ds(row0, rps)])
    copy_d.wait()
    copy_w.wait()
    plsc.subcore_barrier()

    def acc_body(c, _):
        pltpu.sync_copy(w_v.at[c], deg_sh.at[dst_v.at[c]], add=True)
        return 0

    lax.fori_loop(0, nchunks, acc_body, 0)
    plsc.subcore_barrier()
    pltpu.sync_copy(deg_sh.at[pl.ds(row0, rps)], zero_v)
    pltpu.sync_copy(zero_v, out_hbm.at[cid, 0, pl.ds(row0, rps)])


def _deg_partials(dst, w, n_pad, nchunks):
    mesh = plsc.VectorSubcoreMesh(core_axis_name="c", subcore_axis_name="s")
    kern = pl.kernel(
        functools.partial(_deg_body, n_pad=n_pad, nchunks=nchunks),
        out_type=jax.ShapeDtypeStruct((NC, 1, n_pad), jnp.float32),
        mesh=mesh,
        scratch_types=[
            pltpu.VMEM((nchunks, CHUNK), jnp.int32),
            pltpu.VMEM((nchunks, CHUNK), jnp.float32),
            pltpu.VMEM((n_pad // NS,), jnp.float32),
            pltpu.VMEM_SHARED((n_pad,), jnp.float32),
            pltpu.SemaphoreType.DMA,
        ],
    )
    return kern(dst, w)


def _spmm_body(src_hbm, dst_hbm, w_hbm, g_hbm, out_hbm,
               src_v, dst_v, w_v, rows_v, acc_sh, sem,
               g0, g1, g2, g3, s0, s1, s2, s3, *, n_pad, d, nchunks):
    gsems = (g0, g1, g2, g3)
    ssems = (s0, s1, s2, s3)
    cid = lax.axis_index("c")
    sid = lax.axis_index("s")
    wid = sid * NC + cid
    rps = n_pad // NS
    row0 = sid * rps

    copy_s = pltpu.async_copy(src_hbm.at[wid], src_v, sem)
    copy_d = pltpu.async_copy(dst_hbm.at[wid], dst_v, sem)
    copy_w = pltpu.async_copy(w_hbm.at[wid], w_v, sem)

    def zrow(i, _):
        rows_v[0, pl.ds(i, 1), :] = jnp.zeros((1, d), jnp.float32)
        return 0

    lax.fori_loop(0, CHUNK, zrow, 0)
    for k in range(rps // CHUNK):
        pltpu.sync_copy(rows_v.at[0],
                        acc_sh.at[pl.ds(row0 + k * CHUNK, CHUNK)])
    copy_s.wait()
    copy_d.wait()
    copy_w.wait()
    plsc.subcore_barrier()

    nbytes = CHUNK * d * 4

    def gwait(b):
        pltpu.make_async_copy(g_hbm.at[pl.ds(0, CHUNK)], rows_v.at[b],
                              gsems[b]).wait()

    def swait(b):
        pltpu.make_async_copy(g_hbm.at[pl.ds(0, CHUNK)], rows_v.at[b],
                              ssems[b]).wait()

    def round_body(r, _):
        for b in range(NBUF):
            s = r * NBUF + b

            @pl.when(jnp.logical_and(s >= NBUF, s < nchunks))
            def _():
                swait(b)

            @pl.when(s < nchunks)
            def _():
                pltpu.async_copy(g_hbm.at[src_v.at[s]], rows_v.at[b],
                                 gsems[b])

            c = s - 2
            bp = (b + 2) % NBUF

            @pl.when(jnp.logical_and(c >= 0, c < nchunks))
            def _():
                gwait(bp)

                def scale(g, _):
                    w16 = w_v[c, pl.ds(g * LANES, LANES)]
                    base = g * LANES
                    for jj in range(LANES):
                        wj = w16[jj]
                        for k in range(d // LANES):
                            sl = pl.ds(k * LANES, LANES)
                            rows_v[bp, base + jj, sl] = (
                                rows_v[bp, base + jj, sl] * wj)
                    return 0

                lax.fori_loop(0, CHUNK // LANES, scale, 0)
                pltpu.async_copy(rows_v.at[bp], acc_sh.at[dst_v.at[c]],
                                 ssems[bp], add=True)
        return 0

    nrounds = (nchunks + 2 + NBUF - 1) // NBUF
    lax.fori_loop(0, nrounds, round_body, 0)
    for b in range(NBUF):
        swait(b)
    plsc.subcore_barrier()

    for k in range(rps // CHUNK):
        r = row0 + k * CHUNK
        pltpu.sync_copy(acc_sh.at[pl.ds(r, CHUNK)], rows_v.at[0])
        pltpu.sync_copy(rows_v.at[0], out_hbm.at[cid, pl.ds(r, CHUNK)])


def _spmm_partials(src, dst, w, g, n_pad, d, nchunks):
    mesh = plsc.VectorSubcoreMesh(core_axis_name="c", subcore_axis_name="s")
    kern = pl.kernel(
        functools.partial(_spmm_body, n_pad=n_pad, d=d, nchunks=nchunks),
        out_type=jax.ShapeDtypeStruct((NC, n_pad, d), jnp.float32),
        mesh=mesh,
        compiler_params=pltpu.CompilerParams(use_tc_tiling_on_sc=False),
        scratch_types=[
            pltpu.VMEM((nchunks, CHUNK), jnp.int32),
            pltpu.VMEM((nchunks, CHUNK), jnp.int32),
            pltpu.VMEM((nchunks, CHUNK), jnp.float32),
            pltpu.VMEM((NBUF, CHUNK, d), jnp.float32),
            pltpu.VMEM_SHARED((n_pad, d), jnp.float32),
            pltpu.SemaphoreType.DMA,
        ] + [pltpu.SemaphoreType.DMA] * (2 * NBUF),
    )
    return kern(src, dst, w, g)


def _prep_body(degp_ref, x_ref, w1_ref, g1_ref, dinv_ref):
    deg = 1.0 + jnp.sum(degp_ref[...], axis=1)
    dinv = jnp.where(deg > 0, lax.rsqrt(deg), 0.0)
    dinv_ref[...] = dinv[:, None]
    h = jnp.dot(x_ref[...], w1_ref[...], preferred_element_type=jnp.float32)
    g1_ref[...] = h * dinv[:, None]


def _prep(degp, x, w1, n, blk):
    d_in, h_dim = x.shape[1], w1.shape[1]
    grid = (n // blk,)
    return pl.pallas_call(
        _prep_body,
        grid=grid,
        in_specs=[
            pl.BlockSpec((blk, NC), lambda i: (i, 0)),
            pl.BlockSpec((blk, d_in), lambda i: (i, 0)),
            pl.BlockSpec((d_in, h_dim), lambda i: (0, 0)),
        ],
        out_specs=[
            pl.BlockSpec((blk, h_dim), lambda i: (i, 0)),
            pl.BlockSpec((blk, 1), lambda i: (i, 0)),
        ],
        out_shape=[
            jax.ShapeDtypeStruct((n, h_dim), jnp.float32),
            jax.ShapeDtypeStruct((n, 1), jnp.float32),
        ],
    )(degp, x, w1)


def _mid_body(aggp_ref, g1_ref, dinv_ref, b1_ref, w2_ref, g2_ref):
    agg = aggp_ref[0] + aggp_ref[1] + g1_ref[...]
    t = jnp.maximum(agg * dinv_ref[...] + b1_ref[...], 0.0)
    h2 = jnp.dot(t, w2_ref[...], preferred_element_type=jnp.float32)
    g2_ref[...] = h2 * dinv_ref[...]


def _mid(aggp, g1, dinv, b1, w2p, n, blk):
    h_dim, c_pad = w2p.shape
    return pl.pallas_call(
        _mid_body,
        grid=(n // blk,),
        in_specs=[
            pl.BlockSpec((NC, blk, h_dim), lambda i: (0, i, 0)),
            pl.BlockSpec((blk, h_dim), lambda i: (i, 0)),
            pl.BlockSpec((blk, 1), lambda i: (i, 0)),
            pl.BlockSpec((1, h_dim), lambda i: (0, 0)),
            pl.BlockSpec((h_dim, c_pad), lambda i: (0, 0)),
        ],
        out_specs=pl.BlockSpec((blk, c_pad), lambda i: (i, 0)),
        out_shape=jax.ShapeDtypeStruct((n, c_pad), jnp.float32),
    )(aggp, g1, dinv, b1[None, :], w2p)


def _fin_body(aggp_ref, g2_ref, dinv_ref, b2_ref, out_ref, *, c):
    agg = aggp_ref[0] + aggp_ref[1] + g2_ref[...]
    z = (agg * dinv_ref[...] + b2_ref[...])[:, :c]
    m = jnp.max(z, axis=1, keepdims=True)
    lse = jnp.log(jnp.sum(jnp.exp(z - m), axis=1, keepdims=True))
    out_ref[...] = z - m - lse


def _fin(aggp, g2, dinv, b2p, n, c, blk):
    c_pad = g2.shape[1]
    return pl.pallas_call(
        functools.partial(_fin_body, c=c),
        grid=(n // blk,),
        in_specs=[
            pl.BlockSpec((NC, blk, c_pad), lambda i: (0, i, 0)),
            pl.BlockSpec((blk, c_pad), lambda i: (i, 0)),
            pl.BlockSpec((blk, 1), lambda i: (i, 0)),
            pl.BlockSpec((1, c_pad), lambda i: (0, 0)),
        ],
        out_specs=pl.BlockSpec((blk, c), lambda i: (i, 0)),
        out_shape=jax.ShapeDtypeStruct((n, c), jnp.float32),
    )(aggp, g2, dinv, b2p[None, :])


def kernel(x, edge_index, edge_attr, W1, b1, W2, b2):
    n, d_in = x.shape
    e = edge_attr.shape[0]
    h_dim = W1.shape[1]
    c = W2.shape[1]
    c_pad = ((c + LANES - 1) // LANES) * LANES
    n_pad = ((n + NS * CHUNK - 1) // (NS * CHUNK)) * (NS * CHUNK)

    epw = ((e + NW * NBUF * CHUNK - 1) // (NW * NBUF * CHUNK)) * (NBUF * CHUNK)
    e_pad = epw * NW
    nchunks = epw // CHUNK
    pad = e_pad - e
    src = jnp.concatenate([edge_index[0], jnp.zeros((pad,), jnp.int32)])
    dst = jnp.concatenate([edge_index[1], jnp.zeros((pad,), jnp.int32)])
    w = jnp.concatenate([edge_attr, jnp.zeros((pad,), jnp.float32)])
    src3 = src.reshape(NW, nchunks, CHUNK)
    dst3 = dst.reshape(NW, nchunks, CHUNK)
    w3 = w.reshape(NW, nchunks, CHUNK)

    degp = _deg_partials(dst3, w3, n_pad, nchunks)[:, 0, :n].T

    blk = 1000 if n % 1000 == 0 else n // 8
    g1, dinv = _prep(degp, x, W1, n, blk)

    agg1 = _spmm_partials(src3, dst3, w3, g1, n_pad, h_dim, nchunks)[:, :n]

    w2p = jnp.pad(W2, ((0, 0), (0, c_pad - c)))
    b2p = jnp.pad(b2, (0, c_pad - c))
    g2 = _mid(agg1, g1, dinv, b1, w2p, n, blk)

    agg2 = _spmm_partials(src3, dst3, w3, g2, n_pad, c_pad, nchunks)[:, :n]

    return _fin(agg2, g2, dinv, b2p, n, c, blk)
